# baseline (speedup 1.0000x reference)
import jax
import jax.numpy as jnp
from jax import lax
from jax.experimental import pallas as pl
from jax.experimental.pallas import tpu as pltpu
from jax.experimental.pallas import tpu_sc as plsc

VOCAB = 100000
MAX_LEN = 200
EMBED_DIM = 128
BATCH = 4096

NUM_CORES = 2
NUM_SUBCORES = 16
NUM_WORKERS = NUM_CORES * NUM_SUBCORES
TOTAL_ROWS = BATCH * MAX_LEN
ROWS_PER_WORKER = TOTAL_ROWS // NUM_WORKERS
CHUNK = 80
NUM_CHUNKS = ROWS_PER_WORKER // CHUNK
NBUF = 5
POS2 = 240
LANES = 16
LOOKAHEAD = 2


def _sc_body(x_hbm, tok_hbm, pos_hbm, out_hbm, idx_v, rows_v, pos2_v,
             sem_g, sem_o):
    wid = lax.axis_index("s") * NUM_CORES + lax.axis_index("c")
    base_row = pl.multiple_of(wid * ROWS_PER_WORKER, ROWS_PER_WORKER)

    pltpu.sync_copy(x_hbm.at[pl.ds(base_row, ROWS_PER_WORKER)], idx_v)
    pltpu.sync_copy(pos_hbm, pos2_v.at[pl.ds(0, MAX_LEN)])
    pltpu.sync_copy(pos_hbm.at[pl.ds(0, POS2 - MAX_LEN)],
                    pos2_v.at[pl.ds(MAX_LEN, POS2 - MAX_LEN)])

    def gather(c, b):
        off = pl.multiple_of(c * CHUNK, 8)
        return pltpu.make_async_copy(
            tok_hbm.at[idx_v.at[pl.ds(off, CHUNK)]], rows_v.at[b], sem_g)

    def outcp(c, b):
        return pltpu.make_async_copy(
            rows_v.at[b], out_hbm.at[pl.ds(base_row + c * CHUNK, CHUNK)],
            sem_o)

    def add_chunk(l0, b):
        def add_row(i, _):
            for j in range(EMBED_DIM // LANES):
                sl = pl.ds(j * LANES, LANES)
                plsc.addupdate(rows_v.at[b, i, sl], pos2_v[l0 + i, sl])
            return 0

        lax.fori_loop(0, CHUNK, add_row, 0)

    for p in range(LOOKAHEAD):
        gather(p, p).start()

    def ring_body(t, _):
        for k in range(NBUF):
            c = NBUF * t + k
            nb = (k + LOOKAHEAD) % NBUF

            @pl.when(c >= NBUF - LOOKAHEAD)
            def _():
                outcp(c - (NBUF - LOOKAHEAD), nb).wait()

            @pl.when(c + LOOKAHEAD < NUM_CHUNKS)
            def _():
                gather(c + LOOKAHEAD, nb).start()

            gather(c, k).wait()
            add_chunk((CHUNK * k) % MAX_LEN, k)
            outcp(c, k).start()
        return 0

    lax.fori_loop(0, NUM_CHUNKS // NBUF, ring_body, 0)

    for c in range(NUM_CHUNKS - (NBUF - LOOKAHEAD), NUM_CHUNKS):
        outcp(c, c % NBUF).wait()


@jax.jit
def _embed(x1d, token_table, pos_table):
    mesh = plsc.VectorSubcoreMesh(
        core_axis_name="c", subcore_axis_name="s",
        num_cores=NUM_CORES, num_subcores=NUM_SUBCORES)
    fn = pl.kernel(
        _sc_body,
        out_type=jax.ShapeDtypeStruct((TOTAL_ROWS, EMBED_DIM), jnp.float32),
        mesh=mesh,
        scratch_types=[
            pltpu.VMEM((ROWS_PER_WORKER,), jnp.int32),
            pltpu.VMEM((NBUF, CHUNK, EMBED_DIM), jnp.float32),
            pltpu.VMEM((POS2, EMBED_DIM), jnp.float32),
            pltpu.SemaphoreType.DMA,
            pltpu.SemaphoreType.DMA,
        ],
    )
    return fn(x1d, token_table, pos_table)


def kernel(x, token_table, pos_table):
    x1d = x.reshape(TOTAL_ROWS).astype(jnp.int32)
    out = _embed(x1d, token_table, pos_table)
    return out.reshape(BATCH, MAX_LEN, EMBED_DIM)
